# baseline (speedup 1.0000x reference)
import functools
import jax
import jax.numpy as jnp
from jax import lax
from jax.experimental import pallas as pl
from jax.experimental.pallas import tpu as pltpu
from jax.experimental.pallas import tpu_sc as plsc

_QPAD = 5120

_NW = 32
_RW = 2500
_GW = _RW // 16
_WW = _RW * 4


def _sc_boxes(bx_hbm, cls_hbm, ts_hbm, bx_out, cl_out,
              bxv, obv, tsv, clsv, cll):
    wid = lax.axis_index("s") * 2 + lax.axis_index("c")
    iota = lax.iota(jnp.int32, 16)
    base4 = iota * 4

    @pl.when(wid == 0)
    def _():
        pltpu.sync_copy(cls_hbm, clsv)
        m0 = plsc.load_gather(clsv, [iota * 10])
        lab0 = jnp.zeros((16,), jnp.int32)

        def cbody(c, carry):
            m, lab = carry
            v = plsc.load_gather(clsv, [iota * 10 + c])
            upd = v > m
            return jnp.where(upd, v, m), jnp.where(upd, c, lab)

        _, lab = lax.fori_loop(1, 10, cbody, (m0, lab0))
        cll[...] = lab
        pltpu.sync_copy(cll, cl_out)

    pltpu.sync_copy(ts_hbm, tsv)
    pltpu.sync_copy(bx_hbm.at[pl.ds(wid * _WW, _WW)], bxv)

    row_base = wid * _RW

    def group(j, _):
        r0 = jnp.where(j == _GW, _RW - 16, j * 16)
        b4 = base4 + r0 * 4
        row = row_base + r0 + iota
        bidx = (row // 5000) * 2
        shv = plsc.load_gather(tsv, [bidx]).astype(jnp.float32)
        swv = plsc.load_gather(tsv, [bidx + 1]).astype(jnp.float32)
        cx = plsc.load_gather(bxv, [b4])
        cy = plsc.load_gather(bxv, [b4 + 1])
        w = plsc.load_gather(bxv, [b4 + 2])
        h = plsc.load_gather(bxv, [b4 + 3])
        one = jnp.float32(1.0)
        zf = jnp.float32(0.0)
        x0 = jnp.clip(cx - 0.5 * w, zf, one) * swv
        y0 = jnp.clip(cy - 0.5 * h, zf, one) * shv
        x1 = jnp.clip(cx + 0.5 * w, zf, one) * swv
        y1 = jnp.clip(cy + 0.5 * h, zf, one) * shv
        plsc.store_scatter(obv, [b4], x0)
        plsc.store_scatter(obv, [b4 + 1], y0)
        plsc.store_scatter(obv, [b4 + 2], x1)
        plsc.store_scatter(obv, [b4 + 3], y1)
        return 0

    lax.fori_loop(0, _GW + 1, group, 0, unroll=4)
    pltpu.sync_copy(obv, bx_out.at[pl.ds(wid * _WW, _WW)])


def _tc_body(logits_ref, scores_ref, labels_ref):
    nq = logits_ref.shape[1]
    pad = _QPAD - nq
    xt = logits_ref[0].T
    c_iota = jax.lax.broadcasted_iota(jnp.int32, xt.shape, 0)
    m = jnp.max(xt, axis=0)
    labels = jnp.min(jnp.where(xt == m[None, :], c_iota, 91), axis=0)
    s = jnp.sum(jnp.exp(xt), axis=0)
    scores = jnp.exp(m) / s
    scores_ref[0] = jnp.concatenate(
        [scores, jnp.zeros((pad,), jnp.float32)]).reshape(1, _QPAD)
    labels_ref[0] = jnp.concatenate(
        [labels, jnp.zeros((pad,), jnp.int32)]).reshape(1, _QPAD)


def kernel(pred_logits, pred_boxes, cls_logits, target_sizes):
    nb, nq, nc = pred_logits.shape

    scores, labels = pl.pallas_call(
        _tc_body,
        grid=(nb,),
        in_specs=[pl.BlockSpec((1, nq, nc), lambda i: (i, 0, 0))],
        out_specs=[
            pl.BlockSpec((1, 1, _QPAD), lambda i: (i, 0, 0)),
            pl.BlockSpec((1, 1, _QPAD), lambda i: (i, 0, 0)),
        ],
        out_shape=[
            jax.ShapeDtypeStruct((nb, 1, _QPAD), jnp.float32),
            jax.ShapeDtypeStruct((nb, 1, _QPAD), jnp.int32),
        ],
    )(pred_logits)

    mesh = plsc.VectorSubcoreMesh(core_axis_name="c", subcore_axis_name="s")
    sc_fn = functools.partial(
        pl.kernel,
        mesh=mesh,
        compiler_params=pltpu.CompilerParams(needs_layout_passes=False),
        out_type=[
            jax.ShapeDtypeStruct((nb * nq * 4,), jnp.float32),
            jax.ShapeDtypeStruct((nb,), jnp.int32),
        ],
        scratch_types=[
            pltpu.VMEM((_WW,), jnp.float32),
            pltpu.VMEM((_WW,), jnp.float32),
            pltpu.VMEM((32,), jnp.int32),
            pltpu.VMEM((160,), jnp.float32),
            pltpu.VMEM((16,), jnp.int32),
        ],
    )(_sc_boxes)
    boxes, cls2 = sc_fn(pred_boxes.reshape(-1), cls_logits.reshape(-1),
                        target_sizes.reshape(-1))

    return (scores[:, 0, :nq], labels[:, 0, :nq],
            boxes.reshape(nb, nq, 4), cls2)
